# baseline (speedup 1.0000x reference)
import functools

import jax
import jax.numpy as jnp
from jax import lax
from jax.experimental import pallas as pl
from jax.experimental.pallas import tpu as pltpu
from jax.experimental.pallas import tpu_sc as plsc

N = 8388608
N_KNOTS = 10
NSEG = N_KNOTS - 1
NC, NS, L = 2, 16, 16
NW = NC * NS
PER_W = N // NW
CHUNK = 16384
NCHUNK = PER_W // CHUNK
NVEC = CHUNK // L

ALPHA_C = (1.0 / 9.0) / (1.0 / 9.0 + 1e-8)


def _build_tables(w_hbm, wv, y0t, dyt):
    pltpu.sync_copy(w_hbm, wv.at[pl.ds(0, NSEG)])
    lanes = lax.iota(jnp.int32, L)
    valid = lanes < NSEG
    w = jnp.where(valid, wv[...], 1.0)
    u = jnp.exp(w) + 1.0
    y = jnp.maximum(w, 0.7)
    for _ in range(5):
        y = y - 1.0 + u * jnp.exp(-y)
    pw = jnp.where(valid, y, 0.0)
    acc = pw
    for k in (1, 2, 4, 8):
        wv[...] = acc
        g = plsc.load_gather(wv, [jnp.maximum(lanes - k, 0)])
        acc = acc + jnp.where(lanes >= k, g, 0.0)
    wv[...] = acc
    tot = plsc.load_gather(wv, [jnp.full((L,), NSEG - 1, jnp.int32)])
    inv_t = 1.0 / tot
    dy_c = pw * inv_t * jnp.float32(ALPHA_C)
    y0 = (acc - pw) * inv_t
    y0t[...] = y0 - lanes.astype(jnp.float32) * dy_c
    dyt[...] = dy_c


def _spline_vec(xv, y0t, dyt):
    t = xv * jnp.float32(NSEG)
    idx = jnp.minimum(t, jnp.float32(NSEG - 1)).astype(jnp.int32)
    idx = jnp.maximum(idx, 0)
    a = plsc.load_gather(y0t, [idx])
    b = plsc.load_gather(dyt, [idx])
    return a + t * b


def _sc_body(x_hbm, w_hbm, o_hbm, wv, y0t, dyt, xb0, xb1, yb0, yb1,
             si0, si1, so0, so1):
    _build_tables(w_hbm, wv, y0t, dyt)
    wid = lax.axis_index("s") * NC + lax.axis_index("c")
    base = wid * PER_W
    xbs, sis = (xb0, xb1), (si0, si1)
    ybs, sos = (yb0, yb1), (so0, so1)

    def in_copy(c):
        off = pl.multiple_of(base + c * CHUNK, CHUNK)
        return pltpu.make_async_copy(
            x_hbm.at[pl.ds(off, CHUNK)], xbs[c % 2], sis[c % 2])

    def out_copy(c):
        off = pl.multiple_of(base + c * CHUNK, CHUNK)
        return pltpu.make_async_copy(
            ybs[c % 2], o_hbm.at[pl.ds(off, CHUNK)], sos[c % 2])

    in_copy(0).start()
    in_copy(1).start()
    for c in range(NCHUNK):
        in_copy(c).wait()
        if c >= 2:
            out_copy(c - 2).wait()
        xb, yb = xbs[c % 2], ybs[c % 2]

        @plsc.parallel_loop(0, NVEC, unroll=8)
        def _(i):
            sl = pl.ds(i * L, L)
            yb[sl] = _spline_vec(xb[sl], y0t, dyt)

        out_copy(c).start()
        if c + 2 < NCHUNK:
            in_copy(c + 2).start()
    out_copy(NCHUNK - 2).wait()
    out_copy(NCHUNK - 1).wait()


def kernel(x, weights):
    mesh = plsc.VectorSubcoreMesh(core_axis_name="c", subcore_axis_name="s")
    run = functools.partial(
        pl.kernel,
        mesh=mesh,
        out_type=jax.ShapeDtypeStruct((N,), jnp.float32),
        compiler_params=pltpu.CompilerParams(needs_layout_passes=False),
        scratch_types=[
            pltpu.VMEM((L,), jnp.float32),
            pltpu.VMEM((L,), jnp.float32),
            pltpu.VMEM((L,), jnp.float32),
            pltpu.VMEM((CHUNK,), jnp.float32),
            pltpu.VMEM((CHUNK,), jnp.float32),
            pltpu.VMEM((CHUNK,), jnp.float32),
            pltpu.VMEM((CHUNK,), jnp.float32),
            pltpu.SemaphoreType.DMA,
            pltpu.SemaphoreType.DMA,
            pltpu.SemaphoreType.DMA,
            pltpu.SemaphoreType.DMA,
        ],
    )(_sc_body)
    return run(x, weights)


# speedup vs baseline: 1.0115x; 1.0115x over previous
import functools

import jax
import jax.numpy as jnp
from jax import lax
from jax.experimental import pallas as pl
from jax.experimental.pallas import tpu as pltpu
from jax.experimental.pallas import tpu_sc as plsc

N = 8388608
N_KNOTS = 10
NSEG = N_KNOTS - 1
NC, NS, L = 2, 16, 16
NW = NC * NS
PER_W = N // NW
CHUNK = 16384
NCHUNK = PER_W // CHUNK
NVEC = CHUNK // L

ALPHA_C = (1.0 / 9.0) / (1.0 / 9.0 + 1e-8)


def _build_tables(w_hbm, wv, y0t, dyt):
    pltpu.sync_copy(w_hbm, wv.at[pl.ds(0, NSEG)])
    lanes = lax.iota(jnp.int32, L)
    valid = lanes < NSEG
    w = jnp.where(valid, wv[...], 1.0)
    u = jnp.exp(w) + 1.0
    y = jnp.maximum(w, 0.7)
    for _ in range(5):
        y = y - 1.0 + u * jnp.exp(-y)
    pw = jnp.where(valid, y, 0.0)
    acc = pw
    for k in (1, 2, 4, 8):
        wv[...] = acc
        g = plsc.load_gather(wv, [jnp.maximum(lanes - k, 0)])
        acc = acc + jnp.where(lanes >= k, g, 0.0)
    wv[...] = acc
    tot = plsc.load_gather(wv, [jnp.full((L,), NSEG - 1, jnp.int32)])
    inv_t = 1.0 / tot
    dy_c = pw * inv_t * jnp.float32(ALPHA_C)
    y0 = (acc - pw) * inv_t
    y0t[...] = y0 - lanes.astype(jnp.float32) * dy_c
    dyt[...] = dy_c


def _spline_vec(xv, y0t, dyt):
    t = xv * jnp.float32(NSEG)
    idx = jnp.minimum(t, jnp.float32(NSEG - 1)).astype(jnp.int32)
    idx = jnp.maximum(idx, 0)
    a = plsc.load_gather(y0t, [idx])
    b = plsc.load_gather(dyt, [idx])
    return a + t * b


def _sc_body(x_hbm, w_hbm, o_hbm, wv, y0t, dyt, xb0, xb1, yb0, yb1,
             si0, si1, so0, so1):
    wid = lax.axis_index("s") * NC + lax.axis_index("c")
    base = wid * PER_W
    xbs, sis = (xb0, xb1), (si0, si1)
    ybs, sos = (yb0, yb1), (so0, so1)

    def in_copy(c):
        off = pl.multiple_of(base + c * CHUNK, CHUNK)
        return pltpu.make_async_copy(
            x_hbm.at[pl.ds(off, CHUNK)], xbs[c % 2], sis[c % 2])

    def out_copy(c):
        off = pl.multiple_of(base + c * CHUNK, CHUNK)
        return pltpu.make_async_copy(
            ybs[c % 2], o_hbm.at[pl.ds(off, CHUNK)], sos[c % 2])

    in_copy(0).start()
    in_copy(1).start()
    _build_tables(w_hbm, wv, y0t, dyt)
    for c in range(NCHUNK):
        in_copy(c).wait()
        if c >= 2:
            out_copy(c - 2).wait()
        xb, yb = xbs[c % 2], ybs[c % 2]

        @plsc.parallel_loop(0, NVEC, unroll=8)
        def _(i):
            sl = pl.ds(i * L, L)
            yb[sl] = _spline_vec(xb[sl], y0t, dyt)

        out_copy(c).start()
        if c + 2 < NCHUNK:
            in_copy(c + 2).start()
    out_copy(NCHUNK - 2).wait()
    out_copy(NCHUNK - 1).wait()


def kernel(x, weights):
    mesh = plsc.VectorSubcoreMesh(core_axis_name="c", subcore_axis_name="s")
    run = functools.partial(
        pl.kernel,
        mesh=mesh,
        out_type=jax.ShapeDtypeStruct((N,), jnp.float32),
        compiler_params=pltpu.CompilerParams(needs_layout_passes=False),
        scratch_types=[
            pltpu.VMEM((L,), jnp.float32),
            pltpu.VMEM((L,), jnp.float32),
            pltpu.VMEM((L,), jnp.float32),
            pltpu.VMEM((CHUNK,), jnp.float32),
            pltpu.VMEM((CHUNK,), jnp.float32),
            pltpu.VMEM((CHUNK,), jnp.float32),
            pltpu.VMEM((CHUNK,), jnp.float32),
            pltpu.SemaphoreType.DMA,
            pltpu.SemaphoreType.DMA,
            pltpu.SemaphoreType.DMA,
            pltpu.SemaphoreType.DMA,
        ],
    )(_sc_body)
    return run(x, weights)
